# Initial kernel scaffold; baseline (speedup 1.0000x reference)
import jax
import jax.numpy as jnp
from jax.experimental import pallas as pl


def kernel(x, batch, Wqkv, g1, b1, g2, b2, Wout, bout):
    raise NotImplementedError("write your pallas kernel here")


# speedup vs baseline: 3.7705x; 3.7705x over previous
import jax
import jax.numpy as jnp
from jax import lax
from jax.experimental import pallas as pl

N = 32768
DIM = 128
HEADS = 8
DH = 16
INNER = HEADS * DH
NSEG = 16
BLK = 1024
NB = N // BLK
EPS = 1e-6

_INTERPRET = False


def _dot_t(a, b):
    return lax.dot_general(a, b, (((1,), (1,)), ((), ())),
                           preferred_element_type=jnp.float32)


def _dot(a, b):
    return lax.dot_general(a, b, (((1,), (0,)), ((), ())),
                           preferred_element_type=jnp.float32)


def _dot_tl(a, b):
    return lax.dot_general(a, b, (((0,), (0,)), ((), ())),
                           preferred_element_type=jnp.float32)


def _stats_kernel(x_ref, b_ref, Wk_ref, Wv_ref, A_ref,
                  g1_ref, b1_ref, g2_ref, b2_ref, ktv_ref, cnt_ref):
    i = pl.program_id(0)

    @pl.when(i == 0)
    def _init():
        ktv_ref[...] = jnp.zeros_like(ktv_ref)
        cnt_ref[...] = jnp.zeros_like(cnt_ref)

    x = x_ref[...]
    A = A_ref[...]
    k = _dot_t(x, Wk_ref[...])
    v = _dot_t(x, Wv_ref[...])

    def ln(t, g, b):
        m = _dot(t, A)
        var = _dot(t * t, A) - m * m
        return (t - m) * lax.rsqrt(var + EPS) * g + b

    k = ln(k, g1_ref[...], b1_ref[...])
    v = ln(v, g2_ref[...], b2_ref[...])

    bcol = b_ref[...]
    cols = lax.broadcasted_iota(jnp.int32, (1, INNER), 1)
    E = (bcol == cols).astype(jnp.float32)
    cnt_ref[...] += jnp.sum(E, axis=0, keepdims=True)

    for s in range(NSEG):
        vm = v * E[:, s:s + 1]
        ktv_ref[s * INNER:(s + 1) * INNER, :] += _dot_tl(k, vm)


def _apply_kernel(x_ref, b_ref, Wq_ref, ktv_ref, cnt_ref, A_ref,
                  Wout_ref, bout_ref, y_ref):
    x = x_ref[...]
    q = _dot_t(x, Wq_ref[...])
    bcol = b_ref[...]
    cols = lax.broadcasted_iota(jnp.int32, (1, INNER), 1)
    E = (bcol == cols).astype(jnp.float32)
    inv = 1.0 / jnp.maximum(cnt_ref[...], 1.0)
    scale = _dot_t(E, inv)
    qs = q * scale
    bd = A_ref[...] * jnp.float32(DH)
    acc = jnp.zeros_like(q)
    for s in range(NSEG):
        ktv_s = ktv_ref[s * INNER:(s + 1) * INNER, :] * bd
        acc = acc + _dot(qs * E[:, s:s + 1], ktv_s)
    y_ref[...] = _dot_t(acc, Wout_ref[...]) + bout_ref[...]


def kernel(x, batch, Wqkv, g1, b1, g2, b2, Wout, bout):
    xf = x.reshape(N, DIM)
    bcol = batch.astype(jnp.int32).reshape(N, 1)
    Wq = Wqkv[0:INNER]
    Wk = Wqkv[INNER:2 * INNER]
    Wv = Wqkv[2 * INNER:3 * INNER]
    A = jnp.kron(jnp.eye(HEADS, dtype=jnp.float32),
                 jnp.ones((DH, DH), jnp.float32) / DH)
    g1t = jnp.tile(g1, HEADS).reshape(1, INNER)
    b1t = jnp.tile(b1, HEADS).reshape(1, INNER)
    g2t = jnp.tile(g2, HEADS).reshape(1, INNER)
    b2t = jnp.tile(b2, HEADS).reshape(1, INNER)
    bout_r = bout.reshape(1, DIM)

    def full(shape):
        return pl.BlockSpec(shape, lambda i: tuple(0 for _ in shape))

    rowblk = pl.BlockSpec((BLK, DIM), lambda i: (i, 0))
    batblk = pl.BlockSpec((BLK, 1), lambda i: (i, 0))

    ktv, cnt = pl.pallas_call(
        _stats_kernel,
        grid=(NB,),
        in_specs=[rowblk, batblk, full((INNER, DIM)), full((INNER, DIM)),
                  full((DIM, DIM)), full((1, INNER)), full((1, INNER)),
                  full((1, INNER)), full((1, INNER))],
        out_specs=[full((NSEG * INNER, INNER)), full((1, INNER))],
        out_shape=[jax.ShapeDtypeStruct((NSEG * INNER, INNER), jnp.float32),
                   jax.ShapeDtypeStruct((1, INNER), jnp.float32)],
        interpret=_INTERPRET,
    )(xf, bcol, Wk, Wv, A, g1t, b1t, g2t, b2t)

    y = pl.pallas_call(
        _apply_kernel,
        grid=(NB,),
        in_specs=[rowblk, batblk, full((INNER, DIM)),
                  full((NSEG * INNER, INNER)), full((1, INNER)),
                  full((DIM, DIM)), full((DIM, INNER)), full((1, DIM))],
        out_specs=rowblk,
        out_shape=jax.ShapeDtypeStruct((N, DIM), jnp.float32),
        interpret=_INTERPRET,
    )(xf, bcol, Wq, ktv, cnt, A, Wout, bout_r)

    return y.reshape(1, N, DIM)


# speedup vs baseline: 5.2044x; 1.3803x over previous
import jax
import jax.numpy as jnp
from jax import lax
from jax.experimental import pallas as pl
from jax.experimental.pallas import tpu as pltpu

N = 32768
DIM = 128
HEADS = 8
DH = 16
INNER = HEADS * DH
NSEG = 16
BLK = 1024
NB = N // BLK
EPS = 1e-6

_INTERPRET = False


def _dot_t(a, b):
    return lax.dot_general(a, b, (((1,), (1,)), ((), ())),
                           preferred_element_type=jnp.float32)


def _dot(a, b):
    return lax.dot_general(a, b, (((1,), (0,)), ((), ())),
                           preferred_element_type=jnp.float32)


def _dot_tl(a, b):
    return lax.dot_general(a, b, (((0,), (0,)), ((), ())),
                           preferred_element_type=jnp.float32)


def _stats_kernel(x_ref, b_ref, Wkv_ref, A_ref,
                  g1_ref, b1_ref, g2_ref, b2_ref, ktv_ref, cnt_ref):
    i = pl.program_id(0)

    @pl.when(i == 0)
    def _init():
        ktv_ref[...] = jnp.zeros_like(ktv_ref)
        cnt_ref[...] = jnp.zeros_like(cnt_ref)

    x = x_ref[...]
    A = A_ref[...]
    kv = _dot_t(x, Wkv_ref[...])
    k = kv[:, :INNER]
    v = kv[:, INNER:]

    def ln(t, g, b):
        m = _dot(t, A)
        var = _dot(t * t, A) - m * m
        return (t - m) * lax.rsqrt(var + EPS) * g + b

    k = ln(k, g1_ref[...], b1_ref[...])
    v = ln(v, g2_ref[...], b2_ref[...])

    bcol = b_ref[...]
    cols = lax.broadcasted_iota(jnp.int32, (1, INNER), 1)
    E = (bcol == cols).astype(jnp.float32)
    cnt_ref[...] += jnp.sum(E, axis=0, keepdims=True)

    smin = jnp.min(bcol)
    smax = jnp.max(bcol)
    for s in range(NSEG):
        @pl.when((s >= smin) & (s <= smax))
        def _acc(s=s):
            vm = v * E[:, s:s + 1]
            ktv_ref[s * INNER:(s + 1) * INNER, :] += _dot_tl(k, vm)


def _apply_kernel(x_ref, b_ref, Wq_ref, ktv_ref, cnt_ref, A_ref,
                  Wout_ref, bout_ref, y_ref, acc_ref):
    x = x_ref[...]
    q = _dot_t(x, Wq_ref[...])
    bcol = b_ref[...]
    cols = lax.broadcasted_iota(jnp.int32, (1, INNER), 1)
    E = (bcol == cols).astype(jnp.float32)
    inv = 1.0 / jnp.maximum(cnt_ref[...], 1.0)
    scale = _dot_t(E, inv)
    qs = q * scale
    bd = A_ref[...] * jnp.float32(DH)
    acc_ref[...] = jnp.zeros_like(acc_ref)
    smin = jnp.min(bcol)
    smax = jnp.max(bcol)
    for s in range(NSEG):
        @pl.when((s >= smin) & (s <= smax))
        def _acc(s=s):
            ktv_s = ktv_ref[s * INNER:(s + 1) * INNER, :] * bd
            acc_ref[...] += _dot(qs * E[:, s:s + 1], ktv_s)
    y_ref[...] = _dot_t(acc_ref[...], Wout_ref[...]) + bout_ref[...]


def kernel(x, batch, Wqkv, g1, b1, g2, b2, Wout, bout):
    xf = x.reshape(N, DIM)
    bcol = batch.astype(jnp.int32).reshape(N, 1)
    Wq = Wqkv[0:INNER]
    Wkv = Wqkv[INNER:3 * INNER]
    A = jnp.kron(jnp.eye(HEADS, dtype=jnp.float32),
                 jnp.ones((DH, DH), jnp.float32) / DH)
    g1t = jnp.tile(g1, HEADS).reshape(1, INNER)
    b1t = jnp.tile(b1, HEADS).reshape(1, INNER)
    g2t = jnp.tile(g2, HEADS).reshape(1, INNER)
    b2t = jnp.tile(b2, HEADS).reshape(1, INNER)
    bout_r = bout.reshape(1, DIM)

    def full(shape):
        return pl.BlockSpec(shape, lambda i: tuple(0 for _ in shape))

    rowblk = pl.BlockSpec((BLK, DIM), lambda i: (i, 0))
    batblk = pl.BlockSpec((BLK, 1), lambda i: (i, 0))

    ktv, cnt = pl.pallas_call(
        _stats_kernel,
        grid=(NB,),
        in_specs=[rowblk, batblk, full((2 * INNER, DIM)),
                  full((DIM, DIM)), full((1, INNER)), full((1, INNER)),
                  full((1, INNER)), full((1, INNER))],
        out_specs=[full((NSEG * INNER, INNER)), full((1, INNER))],
        out_shape=[jax.ShapeDtypeStruct((NSEG * INNER, INNER), jnp.float32),
                   jax.ShapeDtypeStruct((1, INNER), jnp.float32)],
        interpret=_INTERPRET,
    )(xf, bcol, Wkv, A, g1t, b1t, g2t, b2t)

    y = pl.pallas_call(
        _apply_kernel,
        grid=(NB,),
        in_specs=[rowblk, batblk, full((INNER, DIM)),
                  full((NSEG * INNER, INNER)), full((1, INNER)),
                  full((DIM, DIM)), full((DIM, INNER)), full((1, DIM))],
        out_specs=rowblk,
        out_shape=jax.ShapeDtypeStruct((N, DIM), jnp.float32),
        scratch_shapes=[pltpu.VMEM((BLK, INNER), jnp.float32)],
        interpret=_INTERPRET,
    )(xf, bcol, Wq, ktv, cnt, A, Wout, bout_r)

    return y.reshape(1, N, DIM)
